# Initial kernel scaffold; baseline (speedup 1.0000x reference)
import jax
import jax.numpy as jnp
from jax.experimental import pallas as pl


def kernel(x, edge_index, W1l, b1l, W1r, W2l, b2l, W2r):
    raise NotImplementedError("write your pallas kernel here")


# speedup vs baseline: 6.3203x; 6.3203x over previous
import functools

import jax
import jax.numpy as jnp
from jax import lax
from jax.experimental import pallas as pl
from jax.experimental.pallas import tpu as pltpu
from jax.experimental.pallas import tpu_sc as plsc

N = 10000
D = 128
U = 128

_NC = 2
_NS = 16
_NW = _NC * _NS
CH = 200
NCH = N // CH


def _zero_vmem(buf, rows, cols):
    def zrow(r, carry):
        for k in range(cols // 16):
            buf[r, pl.ds(k * 16, 16)] = jnp.zeros((16,), jnp.float32)
        return carry

    lax.fori_loop(0, rows, zrow, 0)


@functools.partial(
    pl.kernel,
    out_type=(
        jax.ShapeDtypeStruct((N, D), jnp.float32),
        jax.ShapeDtypeStruct((N, D), jnp.float32),
    ),
    mesh=plsc.VectorSubcoreMesh(core_axis_name="c", subcore_axis_name="s"),
    scratch_types=[
        pltpu.VMEM_SHARED((N, D), jnp.float32),
        pltpu.VMEM((CH, D), jnp.float32),
        pltpu.VMEM((U,), jnp.int32),
        pltpu.VMEM((U,), jnp.int32),
        pltpu.VMEM((U, D), jnp.float32),
        pltpu.SemaphoreType.DMA,
    ],
)
def _sc_scatter(x, src1d, dst1d, aggA, aggB,
                shared_agg, zbuf, src_v, dst_v, rows_v, sem):
    c = lax.axis_index("c")
    s = lax.axis_index("s")
    e = dst1d.shape[0]
    nu = e // U

    _zero_vmem(zbuf, CH, D)

    def ztile(t, carry):
        pltpu.sync_copy(zbuf, shared_agg.at[pl.ds(t * CH, CH)])
        return carry

    lax.fori_loop(NCH * s // _NS, NCH * (s + 1) // _NS, ztile, 0)
    plsc.subcore_barrier()

    w = s * _NC + c

    def unit(u, carry):
        base = u * U
        pltpu.sync_copy(src1d.at[pl.ds(base, U)], src_v)
        pltpu.sync_copy(dst1d.at[pl.ds(base, U)], dst_v)
        pltpu.async_copy(x.at[src_v], rows_v, sem).wait()
        pltpu.sync_copy(rows_v, shared_agg.at[dst_v], add=True)
        return carry

    lax.fori_loop(nu * w // _NW, nu * (w + 1) // _NW, unit, 0)
    plsc.subcore_barrier()

    def wtile(t, carry):
        rows = pl.ds(t * CH, CH)

        @pl.when(c == 0)
        def _():
            pltpu.sync_copy(shared_agg.at[rows], aggA.at[rows])

        @pl.when(c == 1)
        def _():
            pltpu.sync_copy(shared_agg.at[rows], aggB.at[rows])

        return carry

    lax.fori_loop(NCH * s // _NS, NCH * (s + 1) // _NS, wtile, 0)


@functools.partial(
    pl.kernel,
    out_type=(
        jax.ShapeDtypeStruct((N, 16), jnp.float32),
        jax.ShapeDtypeStruct((N, 16), jnp.float32),
    ),
    mesh=plsc.VectorSubcoreMesh(core_axis_name="c", subcore_axis_name="s"),
    scratch_types=[
        pltpu.VMEM_SHARED((N, 16), jnp.float32),
        pltpu.VMEM((CH, 16), jnp.float32),
        pltpu.VMEM((U, 16), jnp.float32),
        pltpu.VMEM((U,), jnp.int32),
    ],
    compiler_params=pltpu.CompilerParams(use_tc_tiling_on_sc=False),
)
def _sc_count(dst1d, cntA, cntB, shared_cnt, zbufc, ones_v, dst_v):
    c = lax.axis_index("c")
    s = lax.axis_index("s")
    e = dst1d.shape[0]
    nu = e // U

    _zero_vmem(zbufc, CH, 16)

    def orow(r, carry):
        ones_v[r, pl.ds(0, 16)] = jnp.ones((16,), jnp.float32)
        return carry

    lax.fori_loop(0, U, orow, 0)

    def ztile(t, carry):
        pltpu.sync_copy(zbufc, shared_cnt.at[pl.ds(t * CH, CH)])
        return carry

    lax.fori_loop(NCH * s // _NS, NCH * (s + 1) // _NS, ztile, 0)
    plsc.subcore_barrier()

    w = s * _NC + c

    def unit(u, carry):
        pltpu.sync_copy(dst1d.at[pl.ds(u * U, U)], dst_v)
        pltpu.sync_copy(ones_v, shared_cnt.at[dst_v], add=True)
        return carry

    lax.fori_loop(nu * w // _NW, nu * (w + 1) // _NW, unit, 0)
    plsc.subcore_barrier()

    def wtile(t, carry):
        rows = pl.ds(t * CH, CH)

        @pl.when(c == 0)
        def _():
            pltpu.sync_copy(shared_cnt.at[rows], cntA.at[rows])

        @pl.when(c == 1)
        def _():
            pltpu.sync_copy(shared_cnt.at[rows], cntB.at[rows])

        return carry

    lax.fori_loop(NCH * s // _NS, NCH * (s + 1) // _NS, wtile, 0)


_B = 512


def _dense_body(aA, aB, cA, cB, x, WlT, WrT, b, out):
    cnt = cA[:, 0:1] + cB[:, 0:1]
    inv = 1.0 / jnp.maximum(cnt, 1.0)
    m = (aA[...] + aB[...]) * inv
    z = (jnp.dot(m, WlT[...], preferred_element_type=jnp.float32)
         + jnp.dot(x[...], WrT[...], preferred_element_type=jnp.float32)
         + b[...])
    out[...] = jnp.maximum(z, 0.0)


def _dense_layer(aA, aB, cA, cB, x, WlT, WrT, b):
    grid = (pl.cdiv(N, _B),)
    row128 = pl.BlockSpec((_B, D), lambda i: (i, 0))
    row16 = pl.BlockSpec((_B, 16), lambda i: (i, 0))
    wfull = pl.BlockSpec((D, D), lambda i: (0, 0))
    bfull = pl.BlockSpec((1, D), lambda i: (0, 0))
    return pl.pallas_call(
        _dense_body,
        grid=grid,
        in_specs=[row128, row128, row16, row16, row128, wfull, wfull, bfull],
        out_specs=row128,
        out_shape=jax.ShapeDtypeStruct((N, D), jnp.float32),
    )(aA, aB, cA, cB, x, WlT, WrT, b)


def kernel(x, edge_index, W1l, b1l, W1r, W2l, b2l, W2r):
    src = edge_index[0].astype(jnp.int32)
    dst = edge_index[1].astype(jnp.int32)

    cntA, cntB = _sc_count(dst)
    aggA, aggB = _sc_scatter(x, src, dst)
    h = _dense_layer(aggA, aggB, cntA, cntB, x,
                     W1l.T, W1r.T, b1l.reshape(1, D))
    aggA2, aggB2 = _sc_scatter(h, src, dst)
    out = _dense_layer(aggA2, aggB2, cntA, cntB, h,
                       W2l.T, W2r.T, b2l.reshape(1, D))
    return out


# speedup vs baseline: 10.3810x; 1.6425x over previous
import functools

import jax
import jax.numpy as jnp
from jax import lax
from jax.experimental import pallas as pl
from jax.experimental.pallas import tpu as pltpu
from jax.experimental.pallas import tpu_sc as plsc

N = 10000
D = 128
E = 320000
U = 128
NU = E // U
NB = 2
UB = 8

_NC = 2
_NS = 16
_NW = _NC * _NS
NU_W = -(-NU // _NW)
CH = 40
NCH = N // CH

_SC_PARAMS = pltpu.CompilerParams(use_tc_tiling_on_sc=False)


def _zero_vmem(buf, rows, cols):
    def zrow(r, carry):
        for k in range(cols // 16):
            buf[r, pl.ds(k * 16, 16)] = jnp.zeros((16,), jnp.float32)
        return carry

    lax.fori_loop(0, rows, zrow, 0)


@functools.partial(
    pl.kernel,
    out_type=(
        jax.ShapeDtypeStruct((N, D), jnp.float32),
        jax.ShapeDtypeStruct((N, D), jnp.float32),
    ),
    mesh=plsc.VectorSubcoreMesh(core_axis_name="c", subcore_axis_name="s"),
    scratch_types=[
        pltpu.VMEM_SHARED((N, D), jnp.float32),
        pltpu.VMEM((CH, D), jnp.float32),
        pltpu.VMEM((UB, U), jnp.int32),
        pltpu.VMEM((UB, U), jnp.int32),
        pltpu.VMEM((U, D), jnp.float32),
        pltpu.VMEM((U, D), jnp.float32),
        pltpu.SemaphoreType.DMA((NB,)),
        pltpu.SemaphoreType.DMA((NB,)),
    ],
    compiler_params=_SC_PARAMS,
)
def _sc_scatter(x, src2d, dst2d, aggA, aggB,
                shared_agg, zbuf, src_b, dst_b, rows0, rows1, gsem, ssem):
    rows = (rows0, rows1)
    c = lax.axis_index("c")
    s = lax.axis_index("s")

    _zero_vmem(zbuf, CH, D)

    def ztile(t, carry):
        pltpu.sync_copy(zbuf, shared_agg.at[pl.ds(t * CH, CH)])
        return carry

    lax.fori_loop(NCH * s // _NS, NCH * (s + 1) // _NS, ztile, 0)
    plsc.subcore_barrier()

    w = s * _NC + c
    u0 = 2 * ((NU // 2) * w // _NW)
    u1 = 2 * ((NU // 2) * (w + 1) // _NW)

    def start_gather(j, b):
        pltpu.async_copy(x.at[src_b.at[j]], rows[b], gsem.at[b])

    def start_scatter(j, b):
        pltpu.async_copy(rows[b], shared_agg.at[dst_b.at[j]],
                         ssem.at[b], add=True)

    def wait_gather(b):
        pltpu.make_async_copy(x.at[pl.ds(0, U)], rows[b], gsem.at[b]).wait()

    def wait_scatter(b):
        pltpu.make_async_copy(rows[b], shared_agg.at[pl.ds(0, U)],
                              ssem.at[b]).wait()

    def batch(kb, carry):
        base = u0 + kb * UB
        pltpu.sync_copy(src2d.at[pl.ds(base, UB)], src_b)
        pltpu.sync_copy(dst2d.at[pl.ds(base, UB)], dst_b)

        @pl.when(kb > 0)
        def _():
            wait_scatter(0)

        start_gather(0, 0)

        for j in range(UB):
            u = base + j
            b = j % NB

            @pl.when(u < u1)
            def _(j=j, b=b):
                wait_gather(b)
                start_scatter(j, b)

            if j + 1 < UB:
                j2 = j + 1
                b2 = j2 % NB

                @pl.when(base + j2 < u1)
                def _(j2=j2, b2=b2, first=(j2 < NB)):
                    if first:
                        @pl.when(kb > 0)
                        def _():
                            wait_scatter(b2)
                    else:
                        wait_scatter(b2)
                    start_gather(j2, b2)

        return carry

    lax.fori_loop(0, (u1 - u0 + UB - 1) // UB, batch, 0)

    @pl.when(u0 < u1)
    def _():
        wait_scatter(0)
        wait_scatter(1)

    plsc.subcore_barrier()

    def wtile(t, carry):
        tile = pl.ds(t * CH, CH)

        @pl.when(c == 0)
        def _():
            pltpu.sync_copy(shared_agg.at[tile], aggA.at[tile])

        @pl.when(c == 1)
        def _():
            pltpu.sync_copy(shared_agg.at[tile], aggB.at[tile])

        return carry

    lax.fori_loop(NCH * s // _NS, NCH * (s + 1) // _NS, wtile, 0)


@functools.partial(
    pl.kernel,
    out_type=(
        jax.ShapeDtypeStruct((N, 16), jnp.float32),
        jax.ShapeDtypeStruct((N, 16), jnp.float32),
    ),
    mesh=plsc.VectorSubcoreMesh(core_axis_name="c", subcore_axis_name="s"),
    scratch_types=[
        pltpu.VMEM_SHARED((N, 16), jnp.float32),
        pltpu.VMEM((CH, 16), jnp.float32),
        pltpu.VMEM((U, 16), jnp.float32),
        pltpu.VMEM((NU_W, U), jnp.int32),
        pltpu.SemaphoreType.DMA,
    ],
    compiler_params=_SC_PARAMS,
)
def _sc_count(dst2d, cntA, cntB, shared_cnt, zbufc, ones_v, dst_all, csem):
    c = lax.axis_index("c")
    s = lax.axis_index("s")

    _zero_vmem(zbufc, CH, 16)

    def orow(r, carry):
        ones_v[r, pl.ds(0, 16)] = jnp.ones((16,), jnp.float32)
        return carry

    lax.fori_loop(0, U, orow, 0)

    def ztile(t, carry):
        pltpu.sync_copy(zbufc, shared_cnt.at[pl.ds(t * CH, CH)])
        return carry

    lax.fori_loop(NCH * s // _NS, NCH * (s + 1) // _NS, ztile, 0)
    plsc.subcore_barrier()

    w = s * _NC + c
    u0 = NU * w // _NW
    u1 = NU * (w + 1) // _NW
    pltpu.sync_copy(dst2d.at[pl.ds(u0, NU_W)], dst_all)

    def batch(k, carry):
        base = u0 + k * 8
        for b in range(8):
            @pl.when(base + b < u1)
            def _(u=base + b):
                pltpu.async_copy(ones_v, shared_cnt.at[dst_all.at[u - u0]],
                                 csem, add=True)
        for b in range(8):
            @pl.when(base + b < u1)
            def _():
                pltpu.make_async_copy(ones_v, shared_cnt.at[pl.ds(0, U)],
                                      csem).wait()
        return carry

    lax.fori_loop(0, (u1 - u0 + 7) // 8, batch, 0)
    plsc.subcore_barrier()

    def wtile(t, carry):
        tile = pl.ds(t * CH, CH)

        @pl.when(c == 0)
        def _():
            pltpu.sync_copy(shared_cnt.at[tile], cntA.at[tile])

        @pl.when(c == 1)
        def _():
            pltpu.sync_copy(shared_cnt.at[tile], cntB.at[tile])

        return carry

    lax.fori_loop(NCH * s // _NS, NCH * (s + 1) // _NS, wtile, 0)


_B = 512


def _dense_body(aA, aB, cA, cB, x, WlT, WrT, b, out):
    cnt = cA[:, 0:1] + cB[:, 0:1]
    inv = 1.0 / jnp.maximum(cnt, 1.0)
    m = (aA[...] + aB[...]) * inv
    z = (jnp.dot(m, WlT[...], preferred_element_type=jnp.float32)
         + jnp.dot(x[...], WrT[...], preferred_element_type=jnp.float32)
         + b[...])
    out[...] = jnp.maximum(z, 0.0)


def _dense_layer(aA, aB, cA, cB, x, WlT, WrT, b):
    grid = (pl.cdiv(N, _B),)
    row128 = pl.BlockSpec((_B, D), lambda i: (i, 0))
    row16 = pl.BlockSpec((_B, 16), lambda i: (i, 0))
    wfull = pl.BlockSpec((D, D), lambda i: (0, 0))
    bfull = pl.BlockSpec((1, D), lambda i: (0, 0))
    return pl.pallas_call(
        _dense_body,
        grid=grid,
        in_specs=[row128, row128, row16, row16, row128, wfull, wfull, bfull],
        out_specs=row128,
        out_shape=jax.ShapeDtypeStruct((N, D), jnp.float32),
    )(aA, aB, cA, cB, x, WlT, WrT, b)


def kernel(x, edge_index, W1l, b1l, W1r, W2l, b2l, W2r):
    src2d = jnp.pad(edge_index[0].astype(jnp.int32).reshape(NU, U),
                    ((0, UB), (0, 0)))
    dst2d = jnp.pad(edge_index[1].astype(jnp.int32).reshape(NU, U),
                    ((0, UB), (0, 0)))

    cntA, cntB = _sc_count(dst2d)
    aggA, aggB = _sc_scatter(x, src2d, dst2d)
    h = _dense_layer(aggA, aggB, cntA, cntB, x,
                     W1l.T, W1r.T, b1l.reshape(1, D))
    aggA2, aggB2 = _sc_scatter(h, src2d, dst2d)
    out = _dense_layer(aggA2, aggB2, cntA, cntB, h,
                       W2l.T, W2r.T, b2l.reshape(1, D))
    return out
